# baseline (speedup 1.0000x reference)
import jax
import jax.numpy as jnp
from jax import lax
from jax.experimental import pallas as pl
from jax.experimental.pallas import tpu as pltpu
from jax.experimental.pallas import tpu_sc as plsc

_NUM_CORES = 2
_NUM_SUBCORES = 16
_LANES = 16
_NUM_WORKERS = _NUM_CORES * _NUM_SUBCORES

_TABLE_ROWS = 961
_TABLE_STRIDE = 968
_NUM_HEADS = 32
_N2 = 256 * 256
_POS_PER_WORKER = _N2 // _NUM_WORKERS
_GROUPS = _POS_PER_WORKER // _LANES


def _sc_body(
    table_hbm, idx_hbm, out_hbm, table_v, idx_v, out_v, sem_t, sem_i, sem_o
):
    wid = lax.axis_index("s") * _NUM_CORES + lax.axis_index("c")
    base = wid * _POS_PER_WORKER

    t_copy = pltpu.async_copy(table_hbm, table_v, sem_t)
    i_copy = pltpu.async_copy(idx_hbm.at[pl.ds(base, _POS_PER_WORKER)], idx_v, sem_i)
    i_copy.wait()
    t_copy.wait()

    out_copies = []
    for c in range(2):
        @plsc.parallel_loop(c * (_GROUPS // 2), (c + 1) * (_GROUPS // 2), 1, unroll=4)
        def group(i):
            off = i * _LANES
            ivec = idx_v[pl.ds(off, _LANES)]
            for h in range(_NUM_HEADS):
                vals = plsc.load_gather(
                    table_v.at[pl.ds(h * _TABLE_STRIDE, _TABLE_ROWS)], [ivec]
                )
                out_v[h, i >> 4, pl.ds((i & 15) * _LANES, _LANES)] = vals

        out_copies.append(
            pltpu.async_copy(
                out_v.at[:, pl.ds(c * 4, 4), :],
                out_hbm.at[:, pl.ds(wid * 8 + c * 4, 4), :],
                sem_o,
            )
        )
    for copy in out_copies:
        copy.wait()


def kernel(relative_position_bias_table, relative_position_index):
    table_flat = jnp.pad(
        relative_position_bias_table.T, ((0, 0), (0, _TABLE_STRIDE - _TABLE_ROWS))
    ).reshape(-1)
    idx_flat = relative_position_index.reshape(-1).astype(jnp.int32)

    mesh = plsc.VectorSubcoreMesh(
        core_axis_name="c",
        subcore_axis_name="s",
        num_cores=_NUM_CORES,
        num_subcores=_NUM_SUBCORES,
    )
    out_flat = pl.kernel(
        _sc_body,
        out_type=jax.ShapeDtypeStruct((_NUM_HEADS, 256, 256), jnp.float32),
        mesh=mesh,
        scratch_types=[
            pltpu.VMEM((_TABLE_STRIDE * _NUM_HEADS,), jnp.float32),
            pltpu.VMEM((_POS_PER_WORKER,), jnp.int32),
            pltpu.VMEM((_NUM_HEADS, 8, 256), jnp.float32),
            pltpu.SemaphoreType.DMA,
            pltpu.SemaphoreType.DMA,
            pltpu.SemaphoreType.DMA,
        ],
        compiler_params=pltpu.CompilerParams(
            needs_layout_passes=False, use_tc_tiling_on_sc=True
        ),
        name="relative_position_bias_sc",
    )(table_flat, idx_flat)

    return out_flat


# speedup vs baseline: 1.0228x; 1.0228x over previous
import jax
import jax.numpy as jnp
from jax import lax
from jax.experimental import pallas as pl
from jax.experimental.pallas import tpu as pltpu
from jax.experimental.pallas import tpu_sc as plsc

_NUM_CORES = 2
_NUM_SUBCORES = 16
_LANES = 16
_NUM_WORKERS = _NUM_CORES * _NUM_SUBCORES

_TABLE_ROWS = 961
_TABLE_STRIDE = 968
_NUM_HEADS = 32
_N2 = 256 * 256
_POS_PER_WORKER = _N2 // _NUM_WORKERS
_GROUPS = _POS_PER_WORKER // _LANES


def _sc_body(
    table_hbm, idx_hbm, out_hbm, table_v, idx_v, out_v, sem_t, sem_i, sem_o
):
    wid = lax.axis_index("s") * _NUM_CORES + lax.axis_index("c")
    base = wid * _POS_PER_WORKER

    t_copy = pltpu.async_copy(table_hbm, table_v, sem_t)
    i_copy = pltpu.async_copy(idx_hbm.at[pl.ds(base, _POS_PER_WORKER)], idx_v, sem_i)
    i_copy.wait()
    t_copy.wait()

    out_copies = []
    for c in range(2):
        @plsc.parallel_loop(c * (_GROUPS // 2), (c + 1) * (_GROUPS // 2), 1, unroll=1)
        def group(i):
            off = i * _LANES
            ivec = idx_v[pl.ds(off, _LANES)]
            for h in range(_NUM_HEADS):
                vals = plsc.load_gather(
                    table_v.at[pl.ds(h * _TABLE_STRIDE, _TABLE_ROWS)], [ivec]
                )
                out_v[h, i >> 4, pl.ds((i & 15) * _LANES, _LANES)] = vals

        out_copies.append(
            pltpu.async_copy(
                out_v.at[:, pl.ds(c * 4, 4), :],
                out_hbm.at[:, pl.ds(wid * 8 + c * 4, 4), :],
                sem_o,
            )
        )
    for copy in out_copies:
        copy.wait()


def kernel(relative_position_bias_table, relative_position_index):
    table_flat = jnp.pad(
        relative_position_bias_table.T, ((0, 0), (0, _TABLE_STRIDE - _TABLE_ROWS))
    ).reshape(-1)
    idx_flat = relative_position_index.reshape(-1).astype(jnp.int32)

    mesh = plsc.VectorSubcoreMesh(
        core_axis_name="c",
        subcore_axis_name="s",
        num_cores=_NUM_CORES,
        num_subcores=_NUM_SUBCORES,
    )
    out_flat = pl.kernel(
        _sc_body,
        out_type=jax.ShapeDtypeStruct((_NUM_HEADS, 256, 256), jnp.float32),
        mesh=mesh,
        scratch_types=[
            pltpu.VMEM((_TABLE_STRIDE * _NUM_HEADS,), jnp.float32),
            pltpu.VMEM((_POS_PER_WORKER,), jnp.int32),
            pltpu.VMEM((_NUM_HEADS, 8, 256), jnp.float32),
            pltpu.SemaphoreType.DMA,
            pltpu.SemaphoreType.DMA,
            pltpu.SemaphoreType.DMA,
        ],
        compiler_params=pltpu.CompilerParams(
            needs_layout_passes=False, use_tc_tiling_on_sc=True
        ),
        name="relative_position_bias_sc",
    )(table_flat, idx_flat)

    return out_flat


# speedup vs baseline: 1.0386x; 1.0154x over previous
import jax
import jax.numpy as jnp
from jax import lax
from jax.experimental import pallas as pl
from jax.experimental.pallas import tpu as pltpu
from jax.experimental.pallas import tpu_sc as plsc

_NUM_CORES = 2
_NUM_SUBCORES = 16
_LANES = 16
_NUM_WORKERS = _NUM_CORES * _NUM_SUBCORES

_TABLE_ROWS = 961
_TABLE_STRIDE = 968
_NUM_HEADS = 32
_N2 = 256 * 256
_POS_PER_WORKER = _N2 // _NUM_WORKERS
_GROUPS = _POS_PER_WORKER // _LANES


def _sc_body(
    table_hbm, idx_hbm, out_hbm, table_v, idx_v, out_v, sem_t, sem_i, sem_o
):
    wid = lax.axis_index("s") * _NUM_CORES + lax.axis_index("c")

    t_copy = pltpu.async_copy(table_hbm, table_v, sem_t)
    i_copy = pltpu.async_copy(idx_hbm.at[pl.ds(wid * 8, 8), :], idx_v, sem_i)
    i_copy.wait()
    t_copy.wait()

    out_copies = []
    for c in range(2):
        @plsc.parallel_loop(c * (_GROUPS // 2), (c + 1) * (_GROUPS // 2), 1, unroll=1)
        def group(i):
            off = i * _LANES
            ivec = idx_v[i >> 4, pl.ds((i & 15) * _LANES, _LANES)]
            for h in range(_NUM_HEADS):
                vals = plsc.load_gather(
                    table_v.at[pl.ds(h * _TABLE_STRIDE, _TABLE_ROWS)], [ivec]
                )
                out_v[h, i >> 4, pl.ds((i & 15) * _LANES, _LANES)] = vals

        out_copies.append(
            pltpu.async_copy(
                out_v.at[:, pl.ds(c * 4, 4), :],
                out_hbm.at[:, pl.ds(wid * 8 + c * 4, 4), :],
                sem_o,
            )
        )
    for copy in out_copies:
        copy.wait()


def kernel(relative_position_bias_table, relative_position_index):
    table_flat = jnp.pad(
        relative_position_bias_table.T, ((0, 0), (0, _TABLE_STRIDE - _TABLE_ROWS))
    ).reshape(-1)
    idx = relative_position_index.astype(jnp.int32)

    mesh = plsc.VectorSubcoreMesh(
        core_axis_name="c",
        subcore_axis_name="s",
        num_cores=_NUM_CORES,
        num_subcores=_NUM_SUBCORES,
    )
    out_flat = pl.kernel(
        _sc_body,
        out_type=jax.ShapeDtypeStruct((_NUM_HEADS, 256, 256), jnp.float32),
        mesh=mesh,
        scratch_types=[
            pltpu.VMEM((_TABLE_STRIDE * _NUM_HEADS,), jnp.float32),
            pltpu.VMEM((8, 256), jnp.int32),
            pltpu.VMEM((_NUM_HEADS, 8, 256), jnp.float32),
            pltpu.SemaphoreType.DMA,
            pltpu.SemaphoreType.DMA,
            pltpu.SemaphoreType.DMA,
        ],
        compiler_params=pltpu.CompilerParams(
            needs_layout_passes=False, use_tc_tiling_on_sc=True
        ),
        name="relative_position_bias_sc",
    )(table_flat, idx)

    return out_flat
